# baseline (speedup 1.0000x reference)
import functools

import jax
import jax.numpy as jnp
from jax import lax
from jax.experimental import pallas as pl
from jax.experimental.pallas import tpu as pltpu
from jax.experimental.pallas import tpu_sc as plsc

_VOCAB = 100000
_D = 64
_S = 8
_B = 4096
_SEQ = 52
_L = 16
_NC = 2
_NS = 16
_NW = _NC * _NS
_BPW = _B // _NW
_KD = _D // _L
_NBUF = 2
_WSC = 32
_NQ = _BPW // _WSC


@functools.partial(
    pl.kernel,
    mesh=plsc.VectorSubcoreMesh(core_axis_name="c", subcore_axis_name="s"),
    compiler_params=pltpu.CompilerParams(use_tc_tiling_on_sc=True,
                                         needs_layout_passes=False),
    out_type=jax.ShapeDtypeStruct((_NW, _S, 8 * _L), jnp.float32),
    scratch_types=[
        pltpu.VMEM((_SEQ, _BPW), jnp.int32),
        pltpu.VMEM((_SEQ, _BPW), jnp.int32),
        pltpu.VMEM((_NBUF, _BPW, 2 * _D), jnp.float32),
        pltpu.VMEM((_D, _BPW), jnp.float32),
        pltpu.VMEM((_BPW, _D), jnp.float32),
        pltpu.VMEM((2, _WSC, _S * _D), jnp.float32),
        pltpu.VMEM((_S, 8 * _L), jnp.float32),
        pltpu.SemaphoreType.DMA,
        pltpu.SemaphoreType.DMA,
        pltpu.SemaphoreType.DMA,
    ],
)
def _sense_partials(xT_hbm, wg2_hbm, ws2_hbm, out_hbm,
                    x_v, hx_v, rows_v, acct_v, accb_v, ws_v, part_v,
                    sem_ws, sem0, sem1):
    wid = lax.axis_index("s") * _NC + lax.axis_index("c")
    sems = (sem0, sem1)
    iota = lax.iota(jnp.int32, _L)
    zeros = jnp.zeros((_L,), jnp.float32)

    pltpu.sync_copy(xT_hbm.at[wid], x_v)

    def half_body(c, carry):
        for j in range(_BPW // _L):
            sl = pl.ds(j * _L, _L)
            hx_v[c, sl] = lax.shift_right_logical(x_v[c, sl], 1)
        return carry

    lax.fori_loop(0, _SEQ, half_body, 0)

    for q in range(2):
        pltpu.async_copy(ws2_hbm.at[x_v.at[0, pl.ds(q * _WSC, _WSC)]],
                         ws_v.at[q], sem_ws)

    def start_col(c, buf):
        pltpu.async_copy(wg2_hbm.at[hx_v.at[c]], rows_v.at[buf], sems[buf])

    def wait_col(c, buf):
        pltpu.make_async_copy(
            wg2_hbm.at[hx_v.at[c]], rows_v.at[buf], sems[buf]).wait()

    def acc_col(c, buf, first):
        rbase = [jnp.full((_L,), i0 * _L, jnp.int32) + iota
                 for i0 in range(_BPW // _L)]
        par = [lax.shift_left(
                   jnp.bitwise_and(x_v[c, pl.ds(i0 * _L, _L)], 1), 6)
               for i0 in range(_BPW // _L)]

        def body(d, carry):
            dv = jnp.full((_L,), d, jnp.int32)
            for i0 in range(_BPW // _L):
                col = par[i0] + dv
                g = plsc.load_gather(rows_v.at[buf], [rbase[i0], col])
                sl = pl.ds(i0 * _L, _L)
                if first:
                    acct_v[d, sl] = g
                else:
                    plsc.addupdate(acct_v.at[d, sl], g)
            return carry

        lax.fori_loop(0, _D, body, 0)

    for t in range(_NBUF):
        start_col(2 + t, t)

    wait_col(2, 0)
    acc_col(2, 0, first=True)
    start_col(4, 0)

    def ring_body(j, carry):
        c0 = 3 + 2 * j
        for t in range(2):
            buf = (1 + t) % _NBUF
            c = c0 + t
            wait_col(c, buf)
            acc_col(c, buf, first=False)

            @pl.when(c + _NBUF < _SEQ)
            def _():
                start_col(c + _NBUF, buf)
        return carry

    lax.fori_loop(0, 24, ring_body, 0)

    wait_col(51, 1)
    acc_col(51, 1, first=False)

    def tr_body(i, carry):
        iv = jnp.full((_L,), i, jnp.int32)
        for k in range(_KD):
            rows = jnp.full((_L,), k * _L, jnp.int32) + iota
            accb_v[i, pl.ds(k * _L, _L)] = plsc.load_gather(
                acct_v, [rows, iv])
        return carry

    lax.fori_loop(0, _BPW, tr_body, 0)

    accs = tuple(zeros for _ in range(_S))
    for q in range(_NQ):
        qb = q % 2
        idxref = x_v.at[0, pl.ds(q * _WSC, _WSC)]
        pltpu.make_async_copy(ws2_hbm.at[idxref], ws_v.at[qb], sem_ws).wait()

        def score_body(i, acc_c, q=q, qb=qb):
            ctx = [accb_v[q * _WSC + i, pl.ds(k * _L, _L)]
                   for k in range(_KD)]
            out = []
            for s in range(_S):
                a = acc_c[s]
                for k in range(_KD):
                    a = a + ws_v[qb, i, pl.ds(s * _D + k * _L, _L)] * ctx[k]
                out.append(a)
            return tuple(out)

        accs = lax.fori_loop(0, _WSC, score_body, accs)

        if q + 2 < _NQ:
            nidx = x_v.at[0, pl.ds((q + 2) * _WSC, _WSC)]
            pltpu.async_copy(ws2_hbm.at[nidx], ws_v.at[qb], sem_ws)

    for s in range(_S):
        for k in range(8):
            part_v[s, pl.ds(k * _L, _L)] = accs[s] if k == 0 else zeros
    pltpu.sync_copy(part_v, out_hbm.at[wid])


@jax.jit
def kernel(x, W_g, W_s):
    xT = x.T.reshape(_SEQ, _NW, _BPW).transpose(1, 0, 2)
    wg2 = W_g.reshape(_VOCAB // 2, 2 * _D)
    ws2 = W_s.reshape(_VOCAB, _S * _D)
    partials = _sense_partials(xT, wg2, ws2)
    return jax.nn.sigmoid(jnp.sum(partials, axis=(0, 2)))


# speedup vs baseline: 2.0723x; 2.0723x over previous
import functools

import jax
import jax.numpy as jnp
from jax import lax
from jax.experimental import pallas as pl
from jax.experimental.pallas import tpu as pltpu
from jax.experimental.pallas import tpu_sc as plsc

_VOCAB = 100000
_D = 64
_S = 8
_B = 4096
_SEQ = 52
_L = 16
_NC = 2
_NS = 16
_NW = _NC * _NS
_BPW = _B // _NW
_KD = _D // _L
_NBUF = 3
_WSC = 32
_NQ = _BPW // _WSC


@functools.partial(
    pl.kernel,
    mesh=plsc.VectorSubcoreMesh(core_axis_name="c", subcore_axis_name="s"),
    compiler_params=pltpu.CompilerParams(use_tc_tiling_on_sc=False),
    out_type=jax.ShapeDtypeStruct((_NW, _BPW, _D), jnp.float32),
    scratch_types=[
        pltpu.VMEM((_SEQ, _BPW), jnp.int32),
        pltpu.VMEM((_NBUF, _BPW, _D), jnp.float32),
        pltpu.VMEM((_BPW, _D), jnp.float32),
        pltpu.SemaphoreType.DMA,
        pltpu.SemaphoreType.DMA,
        pltpu.SemaphoreType.DMA,
    ],
)
def _context_sums(xT_hbm, wg_hbm, out_hbm,
                  x_v, rows_v, acc_v, sem0, sem1, sem2):
    wid = lax.axis_index("s") * _NC + lax.axis_index("c")
    sems = (sem0, sem1, sem2)

    pltpu.sync_copy(xT_hbm.at[wid], x_v)

    def start_col(c, buf):
        pltpu.async_copy(wg_hbm.at[x_v.at[c]], rows_v.at[buf], sems[buf])

    def wait_col(c, buf):
        pltpu.make_async_copy(
            wg_hbm.at[x_v.at[c]], rows_v.at[buf], sems[buf]).wait()

    def acc_col(buf, first):
        def body(i, carry):
            for k in range(_KD):
                sl = pl.ds(k * _L, _L)
                v = rows_v[buf, i, sl]
                if first:
                    acc_v[i, sl] = v
                else:
                    plsc.addupdate(acc_v.at[i, sl], v)
            return carry
        lax.fori_loop(0, _BPW, body, 0, unroll=4)

    for t in range(_NBUF):
        start_col(2 + t, t)

    wait_col(2, 0)
    acc_col(0, first=True)
    start_col(5, 0)

    def ring_body(j, carry):
        c0 = 3 + 3 * j
        for t in range(3):
            buf = (1 + t) % _NBUF
            c = c0 + t
            wait_col(c, buf)
            acc_col(buf, first=False)

            @pl.when(c + _NBUF < _SEQ)
            def _():
                start_col(c + _NBUF, buf)
        return carry

    lax.fori_loop(0, 16, ring_body, 0)

    wait_col(51, 1)
    acc_col(1, first=False)

    pltpu.sync_copy(acc_v, out_hbm.at[wid])


@functools.partial(
    pl.kernel,
    mesh=plsc.VectorSubcoreMesh(core_axis_name="c", subcore_axis_name="s"),
    compiler_params=pltpu.CompilerParams(use_tc_tiling_on_sc=True,
                                         needs_layout_passes=False),
    out_type=jax.ShapeDtypeStruct((_NW, _S, 8 * _L), jnp.float32),
    scratch_types=[
        pltpu.VMEM((_BPW,), jnp.int32),
        pltpu.VMEM((_BPW, _D), jnp.float32),
        pltpu.VMEM((2, _WSC, _S * _D), jnp.float32),
        pltpu.VMEM((_S, 8 * _L), jnp.float32),
        pltpu.SemaphoreType.DMA,
    ],
)
def _sense_scores(xT_hbm, acc_hbm, ws2_hbm, out_hbm,
                  x0_v, accb_v, ws_v, part_v, sem_ws):
    wid = lax.axis_index("s") * _NC + lax.axis_index("c")
    zeros = jnp.zeros((_L,), jnp.float32)

    pltpu.sync_copy(xT_hbm.at[wid, 0], x0_v)
    for q in range(2):
        pltpu.async_copy(ws2_hbm.at[x0_v.at[pl.ds(q * _WSC, _WSC)]],
                         ws_v.at[q], sem_ws)
    pltpu.sync_copy(acc_hbm.at[wid], accb_v)

    accs = tuple(zeros for _ in range(_S))
    for q in range(_NQ):
        qb = q % 2
        idxref = x0_v.at[pl.ds(q * _WSC, _WSC)]
        pltpu.make_async_copy(ws2_hbm.at[idxref], ws_v.at[qb], sem_ws).wait()

        def score_body(i, acc_c, q=q, qb=qb):
            ctx = [accb_v[q * _WSC + i, pl.ds(k * _L, _L)]
                   for k in range(_KD)]
            out = []
            for s in range(_S):
                a = acc_c[s]
                for k in range(_KD):
                    a = a + ws_v[qb, i, pl.ds(s * _D + k * _L, _L)] * ctx[k]
                out.append(a)
            return tuple(out)

        accs = lax.fori_loop(0, _WSC, score_body, accs)

        if q + 2 < _NQ:
            nidx = x0_v.at[pl.ds((q + 2) * _WSC, _WSC)]
            pltpu.async_copy(ws2_hbm.at[nidx], ws_v.at[qb], sem_ws)

    for s in range(_S):
        for k in range(8):
            part_v[s, pl.ds(k * _L, _L)] = accs[s] if k == 0 else zeros
    pltpu.sync_copy(part_v, out_hbm.at[wid])


@jax.jit
def kernel(x, W_g, W_s):
    xT = x.T.reshape(_SEQ, _NW, _BPW).transpose(1, 0, 2)
    ws2 = W_s.reshape(_VOCAB, _S * _D)
    acc = _context_sums(xT, W_g)
    partials = _sense_scores(xT, acc, ws2)
    return jax.nn.sigmoid(jnp.sum(partials, axis=(0, 2)))


# speedup vs baseline: 2.1345x; 1.0300x over previous
import functools

import jax
import jax.numpy as jnp
from jax import lax
from jax.experimental import pallas as pl
from jax.experimental.pallas import tpu as pltpu
from jax.experimental.pallas import tpu_sc as plsc

_VOCAB = 100000
_D = 64
_S = 8
_B = 4096
_SEQ = 52
_L = 16
_NC = 2
_NS = 16
_NW = _NC * _NS
_BPW = _B // _NW
_KD = _D // _L
_NBUF = 3
_WSC = 32
_NQ = _BPW // _WSC


@functools.partial(
    pl.kernel,
    mesh=plsc.VectorSubcoreMesh(core_axis_name="c", subcore_axis_name="s"),
    compiler_params=pltpu.CompilerParams(use_tc_tiling_on_sc=False,
                                         needs_layout_passes=False),
    out_type=(jax.ShapeDtypeStruct((_NW, _BPW, 2 * _D), jnp.float32),
              jax.ShapeDtypeStruct((_NW, _BPW), jnp.int32)),
    scratch_types=[
        pltpu.VMEM((_BPW, _SEQ), jnp.int32),
        pltpu.VMEM((_SEQ, _BPW), jnp.int32),
        pltpu.VMEM((_NBUF, _BPW, _D), jnp.float32),
        pltpu.VMEM((_BPW, 2 * _D), jnp.float32),
        pltpu.SemaphoreType.DMA,
        pltpu.SemaphoreType.DMA,
        pltpu.SemaphoreType.DMA,
    ],
)
def _context_sums(x_hbm, wg_hbm, acc_hbm, xw_hbm,
                  xr_v, x_v, rows_v, acc_v, sem0, sem1, sem2):
    wid = lax.axis_index("s") * _NC + lax.axis_index("c")
    sems = (sem0, sem1, sem2)
    iota = lax.iota(jnp.int32, _L)
    zeros = jnp.zeros((_L,), jnp.float32)

    pltpu.sync_copy(x_hbm.at[pl.ds(wid * _BPW, _BPW)], xr_v)

    def tr_body(c, carry):
        cv = jnp.full((_L,), c, jnp.int32)
        for j in range(_BPW // _L):
            rows = jnp.full((_L,), j * _L, jnp.int32) + iota
            x_v[c, pl.ds(j * _L, _L)] = plsc.load_gather(xr_v, [rows, cv])
        return carry

    lax.fori_loop(0, _SEQ, tr_body, 0)

    pltpu.sync_copy(x_v.at[0], xw_hbm.at[wid])

    def start_col(c, buf):
        pltpu.async_copy(wg_hbm.at[x_v.at[c]], rows_v.at[buf], sems[buf])

    def wait_col(c, buf):
        pltpu.make_async_copy(
            wg_hbm.at[x_v.at[c]], rows_v.at[buf], sems[buf]).wait()

    def acc_col(buf, first):
        def body(i, carry):
            for k in range(_KD):
                sl = pl.ds(k * _L, _L)
                v = rows_v[buf, i, sl]
                if first:
                    acc_v[i, sl] = v
                else:
                    plsc.addupdate(acc_v.at[i, sl], v)
            if first:
                for k in range(_KD, 2 * _KD):
                    acc_v[i, pl.ds(k * _L, _L)] = zeros
            return carry
        lax.fori_loop(0, _BPW, body, 0, unroll=4)

    for t in range(_NBUF):
        start_col(2 + t, t)

    wait_col(2, 0)
    acc_col(0, first=True)
    start_col(5, 0)

    def ring_body(j, carry):
        c0 = 3 + 3 * j
        for t in range(3):
            buf = (1 + t) % _NBUF
            c = c0 + t
            wait_col(c, buf)
            acc_col(buf, first=False)

            @pl.when(c + _NBUF < _SEQ)
            def _():
                start_col(c + _NBUF, buf)
        return carry

    lax.fori_loop(0, 16, ring_body, 0)

    wait_col(51, 1)
    acc_col(1, first=False)

    pltpu.sync_copy(acc_v, acc_hbm.at[wid])


@functools.partial(
    pl.kernel,
    mesh=plsc.VectorSubcoreMesh(core_axis_name="c", subcore_axis_name="s"),
    compiler_params=pltpu.CompilerParams(use_tc_tiling_on_sc=True,
                                         needs_layout_passes=False),
    out_type=jax.ShapeDtypeStruct((_NW, _S, 8 * _L), jnp.float32),
    scratch_types=[
        pltpu.VMEM((_BPW,), jnp.int32),
        pltpu.VMEM((_BPW, 2 * _D), jnp.float32),
        pltpu.VMEM((2, _WSC, _S * _D), jnp.float32),
        pltpu.VMEM((_S, 8 * _L), jnp.float32),
        pltpu.SemaphoreType.DMA,
    ],
)
def _sense_scores(xw_hbm, acc_hbm, ws2_hbm, out_hbm,
                  x0_v, accb_v, ws_v, part_v, sem_ws):
    wid = lax.axis_index("s") * _NC + lax.axis_index("c")
    zeros = jnp.zeros((_L,), jnp.float32)

    pltpu.sync_copy(xw_hbm.at[wid], x0_v)
    for q in range(2):
        pltpu.async_copy(ws2_hbm.at[x0_v.at[pl.ds(q * _WSC, _WSC)]],
                         ws_v.at[q], sem_ws)
    pltpu.sync_copy(acc_hbm.at[wid], accb_v)

    accs = tuple(zeros for _ in range(_S))
    for q in range(_NQ):
        qb = q % 2
        idxref = x0_v.at[pl.ds(q * _WSC, _WSC)]
        pltpu.make_async_copy(ws2_hbm.at[idxref], ws_v.at[qb], sem_ws).wait()

        def score_body(i, acc_c, q=q, qb=qb):
            ctx = [accb_v[q * _WSC + i, pl.ds(k * _L, _L)]
                   for k in range(_KD)]
            out = []
            for s in range(_S):
                a = acc_c[s]
                for k in range(_KD):
                    a = a + ws_v[qb, i, pl.ds(s * _D + k * _L, _L)] * ctx[k]
                out.append(a)
            return tuple(out)

        accs = lax.fori_loop(0, _WSC, score_body, accs)

        if q + 2 < _NQ:
            nidx = x0_v.at[pl.ds((q + 2) * _WSC, _WSC)]
            pltpu.async_copy(ws2_hbm.at[nidx], ws_v.at[qb], sem_ws)

    for s in range(_S):
        for k in range(8):
            part_v[s, pl.ds(k * _L, _L)] = accs[s] if k == 0 else zeros
    pltpu.sync_copy(part_v, out_hbm.at[wid])


@jax.jit
def kernel(x, W_g, W_s):
    ws2 = W_s.reshape(_VOCAB, _S * _D)
    acc, xw = _context_sums(x, W_g)
    partials = _sense_scores(xw, acc, ws2)
    return jax.nn.sigmoid(jnp.sum(partials, axis=(0, 2)))
